# baseline (speedup 1.0000x reference)
import functools

import jax
import jax.numpy as jnp
from jax import lax
from jax.experimental import pallas as pl
from jax.experimental.pallas import tpu as pltpu
from jax.experimental.pallas import tpu_sc as plsc

B = 4096
S = 200
D = 64
C = 100
NC = 2
NS = 16
NW = NC * NS
BPW = B // NW
HALF = S // 2


def _issue(tbl, idx_v, buf_v, sem, row, par):
    pltpu.make_async_copy(
        tbl.at[idx_v.at[2 * row]], buf_v.at[par, pl.ds(0, HALF)], sem).start()
    pltpu.make_async_copy(
        tbl.at[idx_v.at[2 * row + 1]], buf_v.at[par, pl.ds(HALF, HALF)], sem).start()


def _wait(tbl, idx_v, buf_v, sem, row, par):
    pltpu.make_async_copy(
        tbl.at[idx_v.at[2 * row]], buf_v.at[par, pl.ds(0, HALF)], sem).wait()
    pltpu.make_async_copy(
        tbl.at[idx_v.at[2 * row + 1]], buf_v.at[par, pl.ds(HALF, HALF)], sem).wait()


def _accum_store(buf_v, offs_v, row_v, out_hbm, base, sem2, row, par):
    z = jnp.zeros((16,), jnp.float32)

    def half(h, accs):
        def body(g, accs):
            accs = list(accs)
            full = g < 6
            start = jnp.where(full, g * 16, 84)
            ov = offs_v[2 * row + h, pl.ds(start, 16)]

            def lane(u, accs):
                o = ov[u]
                k = start + u
                out = list(accs)
                for j in range(4):
                    out[j] = accs[j] + buf_v[par, h * HALF + k,
                                             pl.ds(o + 16 * j, 16)]
                return tuple(out)

            head = tuple(accs)
            for u in range(12):
                head = lane(u, head)
            accs = tuple(jnp.where(full, a, b) for a, b in zip(head, accs))
            for u in range(12, 16):
                accs = lane(u, accs)
            return tuple(accs)

        return lax.fori_loop(0, 7, body, accs)

    accs = half(0, (z, z, z, z))
    accs = half(1, accs)

    @pl.when(row >= 2)
    def _():
        pltpu.make_async_copy(
            row_v.at[par], out_hbm.at[pl.ds(base, 1)], sem2).wait()

    for j in range(4):
        row_v[par, 0, pl.ds(16 * j, 16)] = accs[j]
    pltpu.make_async_copy(
        row_v.at[par], out_hbm.at[pl.ds(base + row, 1)], sem2).start()


@functools.partial(
    pl.kernel,
    mesh=plsc.VectorSubcoreMesh(core_axis_name="c", subcore_axis_name="s"),
    out_type=jax.ShapeDtypeStruct((B, D), jnp.float32),
    scratch_types=[
        pltpu.VMEM((2 * BPW, HALF), jnp.int32),
        pltpu.VMEM((2 * BPW, HALF), jnp.int32),
        pltpu.VMEM((2, S, 2 * D), jnp.float32),
        pltpu.VMEM((2, 1, D), jnp.float32),
        pltpu.SemaphoreType.DMA,
        pltpu.SemaphoreType.DMA,
        pltpu.SemaphoreType.DMA,
    ],
)
def _sc_pool(xh_hbm, xo_hbm, tbl_hbm, out_hbm, idx_v, offs_v, buf_v, row_v,
             sem0, sem1, sem2):
    wid = lax.axis_index("s") * NC + lax.axis_index("c")
    base = wid * BPW
    pltpu.sync_copy(xh_hbm.at[pl.ds(2 * base, 2 * BPW)], idx_v)
    pltpu.sync_copy(xo_hbm.at[pl.ds(2 * base, 2 * BPW)], offs_v)
    _issue(tbl_hbm, idx_v, buf_v, sem0, 0, 0)

    def outer(g, carry):
        r0 = 2 * g
        _issue(tbl_hbm, idx_v, buf_v, sem1, r0 + 1, 1)
        _wait(tbl_hbm, idx_v, buf_v, sem0, r0, 0)
        _accum_store(buf_v, offs_v, row_v, out_hbm, base, sem2, r0, 0)

        @pl.when(r0 + 2 < BPW)
        def _():
            _issue(tbl_hbm, idx_v, buf_v, sem0, r0 + 2, 0)

        _wait(tbl_hbm, idx_v, buf_v, sem1, r0 + 1, 1)
        _accum_store(buf_v, offs_v, row_v, out_hbm, base, sem2, r0 + 1, 1)
        return carry

    lax.fori_loop(0, BPW // 2, outer, 0)
    for par in range(2):
        pltpu.make_async_copy(
            row_v.at[par], out_hbm.at[pl.ds(base, 1)], sem2).wait()


def _head_body(p_ref, w_ref, b_ref, o_ref):
    o_ref[...] = lax.dot_general(
        p_ref[...], w_ref[...], (((1,), (1,)), ((), ())),
        preferred_element_type=jnp.float32) + b_ref[...]


def _tc_head(pooled, W, b2):
    BB = 1024
    return pl.pallas_call(
        _head_body,
        grid=(B // BB,),
        in_specs=[
            pl.BlockSpec((BB, D), lambda i: (i, 0)),
            pl.BlockSpec((C, D), lambda i: (0, 0)),
            pl.BlockSpec((1, C), lambda i: (0, 0)),
        ],
        out_specs=pl.BlockSpec((BB, C), lambda i: (i, 0)),
        out_shape=jax.ShapeDtypeStruct((B, C), jnp.float32),
    )(pooled, W, b2)


@jax.jit
def kernel(x, table, W, b):
    xi = x.astype(jnp.int32)
    xh = (xi >> 1).reshape(2 * B, HALF)
    xo = ((xi & 1) << 6).reshape(2 * B, HALF)
    tbl2 = table.reshape(500000, 2 * D)
    pooled = _sc_pool(xh, xo, tbl2)
    return _tc_head(pooled, W, b.reshape(1, C))


# speedup vs baseline: 2.3013x; 2.3013x over previous
import functools

import jax
import jax.numpy as jnp
from jax import lax
from jax.experimental import pallas as pl
from jax.experimental.pallas import tpu as pltpu
from jax.experimental.pallas import tpu_sc as plsc

B = 4096
S = 200
D = 64
C = 100
V = 1000000
NC = 2
NS = 16
NW = NC * NS
BPW = B // NW
HALF = S // 2

_PBK = 4096
H = 123 * _PBK


def _prep_body(a_ref, b_ref, o_ref):
    o_ref[...] = jnp.concatenate([a_ref[...].T, b_ref[...].T], axis=1)


def _tc_prep(tT):
    return pl.pallas_call(
        _prep_body,
        grid=(H // _PBK,),
        in_specs=[
            pl.BlockSpec((D, _PBK), lambda i: (0, i)),
            pl.BlockSpec((D, _PBK),
                         lambda i: (0, jnp.minimum(i + H // _PBK,
                                                   V // _PBK))),
        ],
        out_specs=pl.BlockSpec((_PBK, 2 * D), lambda i: (i, 0)),
        out_shape=jax.ShapeDtypeStruct((H, 2 * D), jnp.float32),
    )(tT, tT)


def _issue(tbl, idx_v, buf_v, sem, row, par):
    pltpu.make_async_copy(
        tbl.at[idx_v.at[2 * row]], buf_v.at[par, pl.ds(0, HALF)], sem).start()
    pltpu.make_async_copy(
        tbl.at[idx_v.at[2 * row + 1]], buf_v.at[par, pl.ds(HALF, HALF)], sem).start()


def _wait(tbl, idx_v, buf_v, sem, row, par):
    pltpu.make_async_copy(
        tbl.at[idx_v.at[2 * row]], buf_v.at[par, pl.ds(0, HALF)], sem).wait()
    pltpu.make_async_copy(
        tbl.at[idx_v.at[2 * row + 1]], buf_v.at[par, pl.ds(HALF, HALF)], sem).wait()


def _accum_store(buf_v, pool_v, row, par):
    z = jnp.zeros((16,), jnp.float32)

    def body(i, accs):
        accs = list(accs)
        for u in range(2):
            for j in range(4):
                accs[j] = accs[j] + buf_v[par, 2 * i + u, pl.ds(16 * j, 16)]
        return tuple(accs)

    accs = lax.fori_loop(0, S // 2, body, (z, z, z, z))
    for j in range(4):
        pool_v[row, pl.ds(16 * j, 16)] = accs[j]


@functools.partial(
    pl.kernel,
    mesh=plsc.VectorSubcoreMesh(core_axis_name="c", subcore_axis_name="s"),
    out_type=jax.ShapeDtypeStruct((B, D), jnp.float32),
    scratch_types=[
        pltpu.VMEM((2 * BPW, HALF), jnp.int32),
        pltpu.VMEM((2, S, D), jnp.float32),
        pltpu.VMEM((BPW, D), jnp.float32),
        pltpu.SemaphoreType.DMA,
        pltpu.SemaphoreType.DMA,
    ],
    compiler_params=pltpu.CompilerParams(use_tc_tiling_on_sc=False),
)
def _sc_pool(x2_hbm, tbl_hbm, out_hbm, idx_v, buf_v, pool_v, sem0, sem1):
    wid = lax.axis_index("s") * NC + lax.axis_index("c")
    base = wid * BPW
    pltpu.sync_copy(x2_hbm.at[pl.ds(2 * base, 2 * BPW)], idx_v)
    _issue(tbl_hbm, idx_v, buf_v, sem0, 0, 0)

    def outer(g, carry):
        r0 = 2 * g
        _issue(tbl_hbm, idx_v, buf_v, sem1, r0 + 1, 1)
        _wait(tbl_hbm, idx_v, buf_v, sem0, r0, 0)
        _accum_store(buf_v, pool_v, r0, 0)

        @pl.when(r0 + 2 < BPW)
        def _():
            _issue(tbl_hbm, idx_v, buf_v, sem0, r0 + 2, 0)

        _wait(tbl_hbm, idx_v, buf_v, sem1, r0 + 1, 1)
        _accum_store(buf_v, pool_v, r0 + 1, 1)
        return carry

    lax.fori_loop(0, BPW // 2, outer, 0)
    pltpu.sync_copy(pool_v, out_hbm.at[pl.ds(base, BPW)])


def _head_body(p_ref, w_ref, b_ref, o_ref):
    o_ref[...] = lax.dot_general(
        p_ref[...], w_ref[...], (((1,), (1,)), ((), ())),
        preferred_element_type=jnp.float32) + b_ref[...]


def _tc_head(pooled, W, b2):
    BB = 1024
    return pl.pallas_call(
        _head_body,
        grid=(B // BB,),
        in_specs=[
            pl.BlockSpec((BB, D), lambda i: (i, 0)),
            pl.BlockSpec((C, D), lambda i: (0, 0)),
            pl.BlockSpec((1, C), lambda i: (0, 0)),
        ],
        out_specs=pl.BlockSpec((BB, C), lambda i: (i, 0)),
        out_shape=jax.ShapeDtypeStruct((B, C), jnp.float32),
    )(pooled, W, b2)


@jax.jit
def kernel(x, table, W, b):
    xi = x.astype(jnp.int32)
    xr = jnp.where(xi < H, 2 * xi, 2 * (xi - H) + 1).reshape(2 * B, HALF)
    tbl = _tc_prep(table.T).reshape(2 * H, D)
    pooled = _sc_pool(xr, tbl)
    return _tc_head(pooled, W, b.reshape(1, C))


# speedup vs baseline: 2.4918x; 1.0828x over previous
import functools

import jax
import jax.numpy as jnp
from jax import lax
from jax.experimental import pallas as pl
from jax.experimental.pallas import tpu as pltpu
from jax.experimental.pallas import tpu_sc as plsc

B = 4096
S = 200
D = 64
C = 100
V = 1000000
NC = 2
NS = 16
NW = NC * NS
BPW = B // NW
HALF = S // 2

_PBK = 8192
H = 62 * _PBK
_LASTB = (V + _PBK - 1) // _PBK - 1


def _prep_body(a_ref, b_ref, o_ref):
    eye = jnp.eye(D, dtype=jnp.float32)
    at = lax.dot_general(a_ref[...], eye, (((0,), (0,)), ((), ())),
                         preferred_element_type=jnp.float32)
    bt = lax.dot_general(b_ref[...], eye, (((0,), (0,)), ((), ())),
                         preferred_element_type=jnp.float32)
    o_ref[...] = jnp.concatenate([at, bt], axis=1)


def _tc_prep(tT):
    return pl.pallas_call(
        _prep_body,
        grid=(H // _PBK,),
        in_specs=[
            pl.BlockSpec((D, _PBK), lambda i: (0, i)),
            pl.BlockSpec((D, _PBK),
                         lambda i: (0, jnp.minimum(i + H // _PBK, _LASTB))),
        ],
        out_specs=pl.BlockSpec((_PBK, 2 * D), lambda i: (i, 0)),
        out_shape=jax.ShapeDtypeStruct((H, 2 * D), jnp.float32),
    )(tT, tT)


def _issue(tbl, idx_v, buf_v, sem, row, par):
    pltpu.make_async_copy(
        tbl.at[idx_v.at[2 * row]], buf_v.at[par, pl.ds(0, HALF)], sem).start()
    pltpu.make_async_copy(
        tbl.at[idx_v.at[2 * row + 1]], buf_v.at[par, pl.ds(HALF, HALF)], sem).start()


def _wait(tbl, idx_v, buf_v, sem, row, par):
    pltpu.make_async_copy(
        tbl.at[idx_v.at[2 * row]], buf_v.at[par, pl.ds(0, HALF)], sem).wait()
    pltpu.make_async_copy(
        tbl.at[idx_v.at[2 * row + 1]], buf_v.at[par, pl.ds(HALF, HALF)], sem).wait()


def _accum_store(buf_v, pool_v, row, par):
    z = jnp.zeros((16,), jnp.float32)

    def body(i, accs):
        accs = list(accs)
        for u in range(2):
            for j in range(4):
                accs[j] = accs[j] + buf_v[par, 2 * i + u, pl.ds(16 * j, 16)]
        return tuple(accs)

    accs = lax.fori_loop(0, S // 2, body, (z, z, z, z))
    for j in range(4):
        pool_v[row, pl.ds(16 * j, 16)] = accs[j]


@functools.partial(
    pl.kernel,
    mesh=plsc.VectorSubcoreMesh(core_axis_name="c", subcore_axis_name="s"),
    out_type=jax.ShapeDtypeStruct((B, D), jnp.float32),
    scratch_types=[
        pltpu.VMEM((2 * BPW, HALF), jnp.int32),
        pltpu.VMEM((2, S, D), jnp.float32),
        pltpu.VMEM((BPW, D), jnp.float32),
        pltpu.SemaphoreType.DMA,
        pltpu.SemaphoreType.DMA,
    ],
    compiler_params=pltpu.CompilerParams(use_tc_tiling_on_sc=False),
)
def _sc_pool(x2_hbm, tbl_hbm, out_hbm, idx_v, buf_v, pool_v, sem0, sem1):
    wid = lax.axis_index("s") * NC + lax.axis_index("c")
    base = wid * BPW
    pltpu.sync_copy(x2_hbm.at[pl.ds(2 * base, 2 * BPW)], idx_v)
    _issue(tbl_hbm, idx_v, buf_v, sem0, 0, 0)

    def outer(g, carry):
        r0 = 2 * g
        _issue(tbl_hbm, idx_v, buf_v, sem1, r0 + 1, 1)
        _wait(tbl_hbm, idx_v, buf_v, sem0, r0, 0)
        _accum_store(buf_v, pool_v, r0, 0)

        @pl.when(r0 + 2 < BPW)
        def _():
            _issue(tbl_hbm, idx_v, buf_v, sem0, r0 + 2, 0)

        _wait(tbl_hbm, idx_v, buf_v, sem1, r0 + 1, 1)
        _accum_store(buf_v, pool_v, r0 + 1, 1)
        return carry

    lax.fori_loop(0, BPW // 2, outer, 0)
    pltpu.sync_copy(pool_v, out_hbm.at[pl.ds(base, BPW)])


def _head_body(p_ref, w_ref, b_ref, o_ref):
    o_ref[...] = lax.dot_general(
        p_ref[...], w_ref[...], (((1,), (1,)), ((), ())),
        preferred_element_type=jnp.float32) + b_ref[...]


def _tc_head(pooled, W, b2):
    BB = 1024
    return pl.pallas_call(
        _head_body,
        grid=(B // BB,),
        in_specs=[
            pl.BlockSpec((BB, D), lambda i: (i, 0)),
            pl.BlockSpec((C, D), lambda i: (0, 0)),
            pl.BlockSpec((1, C), lambda i: (0, 0)),
        ],
        out_specs=pl.BlockSpec((BB, C), lambda i: (i, 0)),
        out_shape=jax.ShapeDtypeStruct((B, C), jnp.float32),
    )(pooled, W, b2)


@jax.jit
def kernel(x, table, W, b):
    xi = x.astype(jnp.int32)
    xr = jnp.where(xi < H, 2 * xi, 2 * (xi - H) + 1).reshape(2 * B, HALF)
    tbl = _tc_prep(table.T).reshape(2 * H, D)
    pooled = _sc_pool(xr, tbl)
    return _tc_head(pooled, W, b.reshape(1, C))


# speedup vs baseline: 2.5358x; 1.0177x over previous
import functools

import jax
import jax.numpy as jnp
from jax import lax
from jax.experimental import pallas as pl
from jax.experimental.pallas import tpu as pltpu
from jax.experimental.pallas import tpu_sc as plsc

B = 4096
S = 200
D = 64
C = 100
V = 1000000
NC = 2
NS = 16
NW = NC * NS
BPW = B // NW
HALF = S // 2

_PBK = 8192
HQ = 31 * _PBK
_LASTB = (V + _PBK - 1) // _PBK - 1


_PCH = 1024


def _pack_bf16(t):
    ti = lax.bitcast_convert_type(t, jnp.int32)
    bf = lax.shift_right_logical(ti + 0x8000, 16)
    return bf[0:D // 2, :] | lax.shift_left(bf[D // 2:D, :], 16)


def _prep_body(a_ref, b_ref, c_ref, d_ref, o_ref):
    for c in range(_PBK // _PCH):
        sl = pl.ds(c * _PCH, _PCH)
        for m, r in enumerate((a_ref, b_ref, c_ref, d_ref)):
            o_ref[sl, 32 * m:32 * m + 32] = _pack_bf16(r[:, sl]).T


def _tc_prep(tT):
    nb = HQ // _PBK
    return pl.pallas_call(
        _prep_body,
        grid=(nb,),
        in_specs=[
            pl.BlockSpec((D, _PBK),
                         lambda i, m=m: (0, jnp.minimum(i + m * nb, _LASTB)))
            for m in range(4)
        ],
        out_specs=pl.BlockSpec((_PBK, 4 * 32), lambda i: (i, 0)),
        out_shape=jax.ShapeDtypeStruct((HQ, 4 * 32), jnp.int32),
    )(tT, tT, tT, tT)


def _issue(tbl, idx_v, buf_v, sem, row, par):
    pltpu.make_async_copy(
        tbl.at[idx_v.at[2 * row]], buf_v.at[par, pl.ds(0, HALF)], sem).start()
    pltpu.make_async_copy(
        tbl.at[idx_v.at[2 * row + 1]], buf_v.at[par, pl.ds(HALF, HALF)], sem).start()


def _wait(tbl, idx_v, buf_v, sem, row, par):
    pltpu.make_async_copy(
        tbl.at[idx_v.at[2 * row]], buf_v.at[par, pl.ds(0, HALF)], sem).wait()
    pltpu.make_async_copy(
        tbl.at[idx_v.at[2 * row + 1]], buf_v.at[par, pl.ds(HALF, HALF)], sem).wait()


_HIMASK = -65536


def _accum_store(buf_v, pool_v, row, par):
    z = jnp.zeros((16,), jnp.float32)

    def body(i, accs):
        a0, a1, a2, a3 = accs
        for u in range(2):
            v0 = buf_v[par, 2 * i + u, pl.ds(0, 16)]
            v1 = buf_v[par, 2 * i + u, pl.ds(16, 16)]
            a0 = a0 + plsc.bitcast(lax.shift_left(v0, 16), jnp.float32)
            a1 = a1 + plsc.bitcast(lax.shift_left(v1, 16), jnp.float32)
            a2 = a2 + plsc.bitcast(v0 & _HIMASK, jnp.float32)
            a3 = a3 + plsc.bitcast(v1 & _HIMASK, jnp.float32)
        return (a0, a1, a2, a3)

    accs = lax.fori_loop(0, S // 2, body, (z, z, z, z))
    for j in range(4):
        pool_v[row, pl.ds(16 * j, 16)] = accs[j]


@functools.partial(
    pl.kernel,
    mesh=plsc.VectorSubcoreMesh(core_axis_name="c", subcore_axis_name="s"),
    out_type=jax.ShapeDtypeStruct((B, D), jnp.float32),
    scratch_types=[
        pltpu.VMEM((2 * BPW, HALF), jnp.int32),
        pltpu.VMEM((2, S, 32), jnp.int32),
        pltpu.VMEM((BPW, D), jnp.float32),
        pltpu.SemaphoreType.DMA,
        pltpu.SemaphoreType.DMA,
    ],
    compiler_params=pltpu.CompilerParams(use_tc_tiling_on_sc=False,
                                         needs_layout_passes=False),
)
def _sc_pool(x2_hbm, tbl_hbm, out_hbm, idx_v, buf_v, pool_v, sem0, sem1):
    wid = lax.axis_index("s") * NC + lax.axis_index("c")
    base = wid * BPW
    pltpu.sync_copy(x2_hbm.at[pl.ds(2 * base, 2 * BPW)], idx_v)
    _issue(tbl_hbm, idx_v, buf_v, sem0, 0, 0)

    def outer(g, carry):
        r0 = 2 * g
        _issue(tbl_hbm, idx_v, buf_v, sem1, r0 + 1, 1)
        _wait(tbl_hbm, idx_v, buf_v, sem0, r0, 0)
        _accum_store(buf_v, pool_v, r0, 0)

        @pl.when(r0 + 2 < BPW)
        def _():
            _issue(tbl_hbm, idx_v, buf_v, sem0, r0 + 2, 0)

        _wait(tbl_hbm, idx_v, buf_v, sem1, r0 + 1, 1)
        _accum_store(buf_v, pool_v, r0 + 1, 1)
        return carry

    lax.fori_loop(0, BPW // 2, outer, 0)
    pltpu.sync_copy(pool_v, out_hbm.at[pl.ds(base, BPW)])


def _head_body(p_ref, w_ref, b_ref, o_ref):
    o_ref[...] = lax.dot_general(
        p_ref[...], w_ref[...], (((1,), (1,)), ((), ())),
        preferred_element_type=jnp.float32) + b_ref[...]


def _tc_head(pooled, W, b2):
    BB = 1024
    return pl.pallas_call(
        _head_body,
        grid=(B // BB,),
        in_specs=[
            pl.BlockSpec((BB, D), lambda i: (i, 0)),
            pl.BlockSpec((C, D), lambda i: (0, 0)),
            pl.BlockSpec((1, C), lambda i: (0, 0)),
        ],
        out_specs=pl.BlockSpec((BB, C), lambda i: (i, 0)),
        out_shape=jax.ShapeDtypeStruct((B, C), jnp.float32),
    )(pooled, W, b2)


@jax.jit
def kernel(x, table, W, b):
    xi = x.astype(jnp.int32)
    m = xi // HQ
    xr = (4 * (xi - m * HQ) + m).reshape(2 * B, HALF)
    tbl = _tc_prep(table.T).reshape(4 * HQ, 32)
    pooled = _sc_pool(xr, tbl)
    return _tc_head(pooled, W, b.reshape(1, C))
